# Initial kernel scaffold; baseline (speedup 1.0000x reference)
import jax
import jax.numpy as jnp
from jax.experimental import pallas as pl


def kernel(x, gate_w, w1, w2, w3):
    raise NotImplementedError("write your pallas kernel here")


# speedup vs baseline: 1.0488x; 1.0488x over previous
import functools

import jax
import jax.numpy as jnp
from jax.experimental import pallas as pl
from jax.experimental.pallas import tpu as pltpu

DIM = 1024
HIDDEN = 2048
N_EXPERTS = 8
T = 2048
TILE_T = 256


def _gate_tile(x, gate_w, e):
    logits = jax.lax.dot_general(
        x, gate_w, (((1,), (1,)), ((), ())),
        preferred_element_type=jnp.float32)
    iota = jax.lax.broadcasted_iota(jnp.int32, logits.shape, 1)
    m1 = jnp.max(logits, axis=1, keepdims=True)
    i1 = jnp.min(jnp.where(logits == m1, iota, N_EXPERTS), axis=1, keepdims=True)
    masked = jnp.where(iota == i1, -jnp.inf, logits)
    m2 = jnp.max(masked, axis=1, keepdims=True)
    i2 = jnp.min(jnp.where(masked == m2, iota, N_EXPERTS), axis=1, keepdims=True)
    w0 = jax.nn.sigmoid(m1 - m2)
    w1 = 1.0 - w0
    cw = jnp.where(i1 == e, w0, 0.0) + jnp.where(i2 == e, w1, 0.0)
    return cw


def _moe_body(gate_ref, x_ref, w1_ref, w3_ref, w2_ref, out_ref, acc_ref):
    e = pl.program_id(0)
    t = pl.program_id(1)
    sl = pl.ds(t * TILE_T, TILE_T)
    x = x_ref[...]
    cw = _gate_tile(x, gate_ref[...], e)

    xb = x.astype(jnp.bfloat16)
    h1 = jax.lax.dot_general(xb, w1_ref[0], (((1,), (1,)), ((), ())),
                             preferred_element_type=jnp.float32)
    h3 = jax.lax.dot_general(xb, w3_ref[0], (((1,), (1,)), ((), ())),
                             preferred_element_type=jnp.float32)
    h = (h1 * jax.nn.sigmoid(h1) * h3).astype(jnp.bfloat16)
    y = jax.lax.dot_general(h, w2_ref[0], (((1,), (1,)), ((), ())),
                            preferred_element_type=jnp.float32)
    contrib = cw * y

    @pl.when(e == 0)
    def _():
        acc_ref[sl, :] = contrib

    @pl.when(e != 0)
    def _():
        acc_ref[sl, :] += contrib

    @pl.when(e == N_EXPERTS - 1)
    def _():
        out_ref[...] = acc_ref[sl, :]


@jax.jit
def kernel(x, gate_w, w1, w2, w3):
    w1b = w1.astype(jnp.bfloat16)
    w3b = w3.astype(jnp.bfloat16)
    w2b = w2.astype(jnp.bfloat16)
    n_t = T // TILE_T
    out = pl.pallas_call(
        _moe_body,
        grid=(N_EXPERTS, n_t),
        in_specs=[
            pl.BlockSpec((N_EXPERTS, DIM), lambda e, t: (0, 0)),
            pl.BlockSpec((TILE_T, DIM), lambda e, t: (t, 0)),
            pl.BlockSpec((1, HIDDEN, DIM), lambda e, t: (e, 0, 0)),
            pl.BlockSpec((1, HIDDEN, DIM), lambda e, t: (e, 0, 0)),
            pl.BlockSpec((1, DIM, HIDDEN), lambda e, t: (e, 0, 0)),
        ],
        out_specs=pl.BlockSpec((TILE_T, DIM), lambda e, t: (t, 0)),
        out_shape=jax.ShapeDtypeStruct((T, DIM), jnp.float32),
        scratch_shapes=[pltpu.VMEM((T, DIM), jnp.float32)],
        compiler_params=pltpu.CompilerParams(
            dimension_semantics=("arbitrary", "arbitrary")),
    )(gate_w, x, w1b, w3b, w2b)
    return out


# speedup vs baseline: 1.2964x; 1.2361x over previous
import functools

import jax
import jax.numpy as jnp
from jax import lax
from jax.experimental import pallas as pl
from jax.experimental.pallas import tpu as pltpu
from jax.experimental.pallas import tpu_sc as plsc

DIM = 1024
HIDDEN = 2048
N_EXPERTS = 8
T = 2048
NPAIR = 2 * T
TILE_M = 256
N_TILES = NPAIR // TILE_M + N_EXPERTS
P = N_TILES * TILE_M
NC, NS = 2, 16
NW = NC * NS
CH = 64
CT = 32


def _route_body(x_ref, gate_ref, pos_ref, wgt_ref, te_ref):
    x = x_ref[...]
    logits = lax.dot_general(x, gate_ref[...], (((1,), (1,)), ((), ())),
                             preferred_element_type=jnp.float32)
    ei = lax.broadcasted_iota(jnp.int32, (T, N_EXPERTS), 1)
    m1 = jnp.max(logits, axis=1, keepdims=True)
    i1 = jnp.min(jnp.where(logits == m1, ei, N_EXPERTS), axis=1, keepdims=True)
    masked = jnp.where(ei == i1, -jnp.inf, logits)
    m2 = jnp.max(masked, axis=1, keepdims=True)
    i2 = jnp.min(jnp.where(masked == m2, ei, N_EXPERTS), axis=1, keepdims=True)
    w0 = jax.nn.sigmoid(m1 - m2)

    sel = jnp.concatenate([i1, i2], axis=0)
    ei2 = lax.broadcasted_iota(jnp.int32, (NPAIR, N_EXPERTS), 1)
    onehot = (ei2 == sel).astype(jnp.int32)
    csum = onehot
    sh = 1
    while sh < NPAIR:
        csum = csum + jnp.concatenate(
            [jnp.zeros((sh, N_EXPERTS), jnp.int32), csum[:NPAIR - sh, :]], axis=0)
        sh *= 2
    counts = csum[NPAIR - 1:NPAIR, :]
    rank = jnp.sum(onehot * csum, axis=1, keepdims=True) - 1
    padded = ((counts + TILE_M - 1) // TILE_M) * TILE_M
    padf = padded.astype(jnp.float32)
    r8 = lax.broadcasted_iota(jnp.int32, (N_EXPERTS, N_EXPERTS), 0)
    c8 = lax.broadcasted_iota(jnp.int32, (N_EXPERTS, N_EXPERTS), 1)
    lt = (r8 < c8).astype(jnp.float32)
    starts = lax.dot_general(padf, lt, (((1,), (0,)), ((), ())),
                             precision=lax.Precision.HIGHEST)
    ends = starts + padf
    eye = (r8 == c8).astype(jnp.float32)
    ends_col = lax.dot_general(eye, ends, (((1,), (1,)), ((), ())),
                               precision=lax.Precision.HIGHEST)
    jt = (lax.broadcasted_iota(jnp.int32, (N_EXPERTS, 128), 1) * TILE_M
          ).astype(jnp.float32)
    te = jnp.sum((ends_col <= jt).astype(jnp.int32), axis=0, keepdims=True)
    te_ref[...] = jnp.minimum(te, N_EXPERTS - 1)

    starts_pair = jnp.sum(onehot.astype(jnp.float32) * starts, axis=1,
                          keepdims=True)
    pos_ref[...] = starts_pair.astype(jnp.int32) + rank
    wpair = jnp.concatenate([w0, 1.0 - w0], axis=0)
    wgt_ref[...] = jnp.broadcast_to(wpair, (NPAIR, 16))


def _route(x, gate_w):
    return pl.pallas_call(
        _route_body,
        out_shape=[
            jax.ShapeDtypeStruct((NPAIR, 1), jnp.int32),
            jax.ShapeDtypeStruct((NPAIR, 16), jnp.float32),
            jax.ShapeDtypeStruct((1, 128), jnp.int32),
        ],
    )(x, gate_w)


def _gather_body(tok_hbm, pos_hbm, x_hbm, xs_hbm, tok_v, pos_v, rows_v, sem):
    wid = lax.axis_index("s") * NC + lax.axis_index("c")
    npw = NPAIR // NW
    for cidx in range(npw // CH):
        base = wid * npw + cidx * CH
        pltpu.sync_copy(tok_hbm.at[pl.ds(base, CH)], tok_v)
        pltpu.sync_copy(pos_hbm.at[pl.ds(base, CH)], pos_v)
        pltpu.async_copy(x_hbm.at[tok_v], rows_v, sem).wait()
        pltpu.async_copy(rows_v, xs_hbm.at[pos_v], sem).wait()


def _gather(tok, pos, x):
    return pl.kernel(
        _gather_body,
        out_type=jax.ShapeDtypeStruct((P, DIM), jnp.float32),
        mesh=plsc.VectorSubcoreMesh(core_axis_name="c", subcore_axis_name="s"),
        scratch_types=[
            pltpu.VMEM((CH,), jnp.int32),
            pltpu.VMEM((CH,), jnp.int32),
            pltpu.VMEM((CH, DIM), jnp.float32),
            pltpu.SemaphoreType.DMA,
        ],
    )(tok, pos, x)


def _ffn_body(te_ref, xs_ref, w1_ref, w3_ref, w2_ref, ys_ref):
    xb = xs_ref[...].astype(jnp.bfloat16)
    h1 = lax.dot_general(xb, w1_ref[0], (((1,), (1,)), ((), ())),
                         preferred_element_type=jnp.float32)
    h3 = lax.dot_general(xb, w3_ref[0], (((1,), (1,)), ((), ())),
                         preferred_element_type=jnp.float32)
    h = (h1 * jax.nn.sigmoid(h1) * h3).astype(jnp.bfloat16)
    ys_ref[...] = lax.dot_general(h, w2_ref[0], (((1,), (1,)), ((), ())),
                                  preferred_element_type=jnp.float32)


def _ffn(te, xs, w1b, w3b, w2b):
    grid_spec = pltpu.PrefetchScalarGridSpec(
        num_scalar_prefetch=1,
        grid=(N_TILES,),
        in_specs=[
            pl.BlockSpec((TILE_M, DIM), lambda j, te: (j, 0)),
            pl.BlockSpec((1, HIDDEN, DIM), lambda j, te: (te[j], 0, 0)),
            pl.BlockSpec((1, HIDDEN, DIM), lambda j, te: (te[j], 0, 0)),
            pl.BlockSpec((1, DIM, HIDDEN), lambda j, te: (te[j], 0, 0)),
        ],
        out_specs=pl.BlockSpec((TILE_M, DIM), lambda j, te: (j, 0)),
    )
    return pl.pallas_call(
        _ffn_body,
        grid_spec=grid_spec,
        out_shape=jax.ShapeDtypeStruct((P, DIM), jnp.float32),
        compiler_params=pltpu.CompilerParams(
            dimension_semantics=("arbitrary",)),
    )(te, xs, w1b, w3b, w2b)


def _combine_body(pos_hbm, wgt_hbm, ys_hbm, out_hbm,
                  pa_v, pb_v, wa_v, wb_v, ya_v, yb_v, o_v, sem):
    wid = lax.axis_index("s") * NC + lax.axis_index("c")
    tpw = T // NW
    for cidx in range(tpw // CT):
        base = wid * tpw + cidx * CT
        pltpu.sync_copy(pos_hbm.at[pl.ds(base, CT)], pa_v)
        pltpu.sync_copy(pos_hbm.at[pl.ds(T + base, CT)], pb_v)
        pltpu.sync_copy(wgt_hbm.at[pl.ds(base, CT)], wa_v)
        pltpu.sync_copy(wgt_hbm.at[pl.ds(T + base, CT)], wb_v)
        pltpu.async_copy(ys_hbm.at[pa_v], ya_v, sem).wait()
        pltpu.async_copy(ys_hbm.at[pb_v], yb_v, sem).wait()

        def tok_body(tk, _):
            wa = wa_v[tk, :]
            wb = wb_v[tk, :]

            def j_body(j, _):
                a = ya_v[tk, pl.ds(j * 16, 16)]
                b = yb_v[tk, pl.ds(j * 16, 16)]
                o_v[tk, pl.ds(j * 16, 16)] = wa * a + wb * b
                return 0

            lax.fori_loop(0, DIM // 16, j_body, 0, unroll=8)
            return 0

        lax.fori_loop(0, CT, tok_body, 0)
        pltpu.sync_copy(o_v, out_hbm.at[pl.ds(base, CT)])


def _combine(pos, wgt, ys):
    return pl.kernel(
        _combine_body,
        out_type=jax.ShapeDtypeStruct((T, DIM), jnp.float32),
        mesh=plsc.VectorSubcoreMesh(core_axis_name="c", subcore_axis_name="s"),
        scratch_types=[
            pltpu.VMEM((CT,), jnp.int32),
            pltpu.VMEM((CT,), jnp.int32),
            pltpu.VMEM((CT, 16), jnp.float32),
            pltpu.VMEM((CT, 16), jnp.float32),
            pltpu.VMEM((CT, DIM), jnp.float32),
            pltpu.VMEM((CT, DIM), jnp.float32),
            pltpu.VMEM((CT, DIM), jnp.float32),
            pltpu.SemaphoreType.DMA,
        ],
    )(pos, wgt, ys)


@jax.jit
def kernel(x, gate_w, w1, w2, w3):
    w1b = w1.astype(jnp.bfloat16)
    w3b = w3.astype(jnp.bfloat16)
    w2b = w2.astype(jnp.bfloat16)
    pos2, wgt, te128 = _route(x, gate_w)
    pos = pos2.reshape(NPAIR)
    te = te128[0, :N_TILES]
    tok = jnp.concatenate([jnp.arange(T, dtype=jnp.int32)] * 2)
    xs = _gather(tok, pos, x)
    ys = _ffn(te, xs, w1b, w3b, w2b)
    return _combine(pos, wgt, ys)
